# baseline (speedup 1.0000x reference)
import functools

import jax
import jax.numpy as jnp
from jax import lax
from jax.experimental import pallas as pl
from jax.experimental.pallas import tpu as pltpu
from jax.experimental.pallas import tpu_sc as plsc

F32 = jnp.float32
D = 128
N_NODES = 10000
E_EDGES = 320000
LN_EPS = 1e-5

NB = 2000
E_PAD = 327680
EB = 2048
GRID_E = E_PAD // EB
OB = EB // 128

NWORK = 32
EPW = E_PAD // NWORK
GCH = 80
NCH = EPW // GCH

SM_W = 128
SM_ROWS = E_PAD // SM_W
SM_NW = 16
SM_RPT = SM_ROWS // SM_NW
NSEG = 10240


def _leaky(x):
    return jnp.where(x >= 0, x, 0.2 * x)


def _ln(x, g, b):
    mu = jnp.mean(x, axis=1, keepdims=True)
    xc = x - mu
    var = jnp.mean(xc * xc, axis=1, keepdims=True)
    return xc / jnp.sqrt(var + LN_EPS) * g + b


def _rnd16(x):
    b = lax.bitcast_convert_type(x, jnp.int32)
    r = b + jnp.int32(0x7FFF) + (lax.shift_right_logical(b, jnp.int32(16))
                                 & jnp.int32(1))
    return lax.shift_right_logical(r, jnp.int32(16))


def _pack2(x_lo, x_hi):
    return _rnd16(x_lo) | lax.shift_left(_rnd16(x_hi), jnp.int32(16))


def _unpack_lo(p):
    return lax.bitcast_convert_type(lax.shift_left(p, jnp.int32(16)), F32)


def _unpack_hi(p):
    return lax.bitcast_convert_type(p & jnp.int32(-65536), F32)


def _pre_body(x_ref, wT_ref, b_ref, wff0T_ref, wq1T_ref, bff0_ref, bq1_ref,
              q0_ref, ta_ref, wfoldT_ref, bfold_ref):
    xb = x_ref[...]

    def proj(j):
        return jnp.dot(xb, wT_ref[j], preferred_element_type=F32) + b_ref[j:j + 1]

    q0_ref[...] = proj(0)
    ta_ref[:, :D] = _pack2(proj(1), proj(2))
    ta_ref[:, D:] = _pack2(proj(3), proj(4))

    @pl.when(pl.program_id(0) == 0)
    def _():
        wfoldT_ref[...] = jnp.dot(wff0T_ref[...], wq1T_ref[...],
                                  preferred_element_type=F32)
        bfold_ref[...] = (jnp.dot(bff0_ref[...], wq1T_ref[...],
                                  preferred_element_type=F32) + bq1_ref[...])


def _col_to_rows(a):
    return jnp.sum(a.reshape(OB, 128, D), axis=2)


def _rows_to_col(w):
    return jnp.broadcast_to(w[:, :, None], (OB, 128, D)).reshape(EB, D)


def _alpha_body(qd_ref, g1_ref, alpha_ref):
    k0 = _unpack_lo(g1_ref[...])
    alpha_ref[...] = _col_to_rows(qd_ref[...] * k0)


def _pass1_body(qd_ref, g1_ref, g2_ref, w_ref, gag_ref, gab_ref,
                wfoldT_ref, bfold_ref, q1_ref, alpha_ref):
    v0 = _unpack_hi(g1_ref[...])
    attn = qd_ref[...] + _rows_to_col(w_ref[...]) * v0
    h = _ln(attn, gag_ref[...], gab_ref[...])
    q1 = (jnp.dot(h.astype(jnp.bfloat16), wfoldT_ref[...].astype(jnp.bfloat16),
                  preferred_element_type=F32) + bfold_ref[...])
    q1_ref[...] = q1
    k1 = _unpack_lo(g2_ref[...])
    alpha_ref[...] = _col_to_rows(q1 * k1)


def _pass2_body(q1_ref, vs_ref, w_ref, gag_ref, gab_ref, wffT_ref, bff_ref,
                gfg_ref, gfb_ref, w1T_ref, b1_ref, w4T_ref, b4_ref,
                w5T_ref, b5_ref, fng_ref, fnb_ref, wvec_ref, bvec_ref,
                out_ref):
    bf16 = jnp.bfloat16
    attn = q1_ref[...] + _rows_to_col(w_ref[...]) * _unpack_hi(vs_ref[...])
    h = _ln(attn, gag_ref[...], gab_ref[...])
    t = (jnp.dot(h.astype(bf16), wffT_ref[...].astype(bf16),
                 preferred_element_type=F32) + bff_ref[...])
    g = _ln(_leaky(t), gfg_ref[...], gfb_ref[...])
    a = _leaky(jnp.dot(g.astype(bf16), w1T_ref[...].astype(bf16),
                       preferred_element_type=F32) + b1_ref[...])
    a = _leaky(jnp.dot(a.astype(bf16), w4T_ref[...].astype(bf16),
                       preferred_element_type=F32) + b4_ref[...])
    a = (jnp.dot(a.astype(bf16), w5T_ref[...].astype(bf16),
                 preferred_element_type=F32) + b5_ref[...] + g)
    u = _ln(a, fng_ref[...], fnb_ref[...])
    out_ref[...] = _col_to_rows(u * wvec_ref[...]) + bvec_ref[...]


def _gather_body(ta_hbm, q0_hbm, src_hbm, dst_hbm,
                 g12_hbm, qd_hbm,
                 idx_s, idx_d, rows_a, rows_q, sem):
    wid = lax.axis_index("s") * 2 + lax.axis_index("c")
    base = wid * EPW

    def step(c, carry):
        off = base + c * GCH
        pltpu.sync_copy(src_hbm.at[pl.ds(off, GCH)], idx_s)
        pltpu.sync_copy(dst_hbm.at[pl.ds(off, GCH)], idx_d)
        cp1 = pltpu.async_copy(ta_hbm.at[idx_s], rows_a, sem)
        cp2 = pltpu.async_copy(q0_hbm.at[idx_d], rows_q, sem)
        cp1.wait()
        cp2.wait()
        pltpu.sync_copy(rows_a, g12_hbm.at[pl.ds(off, GCH)])
        pltpu.sync_copy(rows_q, qd_hbm.at[pl.ds(off, GCH)])
        return carry

    lax.fori_loop(0, NCH, step, 0)


def _softmax_body(alpha_hbm, dst_hbm, w_hbm,
                  al_v, dst_v, s_loc, zb, mxv, mx_all, s_sh, sem):
    cid = lax.axis_index("c")

    @pl.when(cid == 0)
    def _core0():
        _softmax_work(alpha_hbm, dst_hbm, w_hbm,
                      al_v, dst_v, s_loc, zb, mxv, mx_all, s_sh, sem)


def _softmax_work(alpha_hbm, dst_hbm, w_hbm,
                  al_v, dst_v, s_loc, zb, mxv, mx_all, s_sh, sem):
    sid = lax.axis_index("s")
    rbase = sid * SM_RPT
    pltpu.sync_copy(alpha_hbm.at[pl.ds(rbase, SM_RPT)], al_v)
    pltpu.sync_copy(dst_hbm.at[pl.ds(rbase, SM_RPT)], dst_v)

    def zstep(i, c):
        zb[pl.ds(i * 16, 16)] = jnp.zeros((16,), F32)
        return c
    lax.fori_loop(0, 40, zstep, 0)
    pltpu.sync_copy(zb, s_sh.at[pl.ds(sid * 640, 640)])

    def mstep(r, m):
        for k in range(SM_W // 16):
            m = jnp.maximum(m, al_v[r, pl.ds(k * 16, 16)])
        return m
    mvec = lax.fori_loop(0, SM_RPT, mstep, jnp.full((16,), -1e30, F32))
    lanes = lax.iota(jnp.int32, 16)
    dnums = lax.GatherDimensionNumbers(
        offset_dims=(), collapsed_slice_dims=(0,), start_index_map=(0,))
    for sh in (8, 4, 2, 1):
        perm = lax.gather(mvec, (lanes ^ sh)[:, None], dnums, (1,),
                          mode=lax.GatherScatterMode.PROMISE_IN_BOUNDS)
        mvec = jnp.maximum(mvec, perm)
    mxv[...] = mvec
    pltpu.sync_copy(mxv, s_sh.at[pl.ds(NSEG + sid * 16, 16)])
    plsc.subcore_barrier()

    pltpu.sync_copy(s_sh.at[pl.ds(NSEG, 16 * SM_NW)], mx_all)

    def rstep(i, m):
        return jnp.maximum(m, mx_all[pl.ds(i * 16, 16)])
    cvec = lax.fori_loop(0, SM_NW, rstep, jnp.full((16,), -1e30, F32))

    def pstep(r, c):
        for k in range(SM_W // 16):
            a = al_v[r, pl.ds(k * 16, 16)]
            al_v[r, pl.ds(k * 16, 16)] = jnp.exp(a - cvec)
        return c
    lax.fori_loop(0, SM_RPT, pstep, 0)

    def sstep(r, c):
        pltpu.sync_copy(al_v.at[r], s_sh.at[dst_v.at[r]], add=True)
        return c
    lax.fori_loop(0, SM_RPT, sstep, 0)
    plsc.subcore_barrier()

    pltpu.sync_copy(s_sh.at[pl.ds(0, NSEG)], s_loc)

    def wstep(r, c):
        for k in range(SM_W // 16):
            dd = dst_v[r, pl.ds(k * 16, 16)]
            sv = plsc.load_gather(s_loc, [dd])
            p = al_v[r, pl.ds(k * 16, 16)]
            al_v[r, pl.ds(k * 16, 16)] = p / (sv + 1e-16)
        return c
    lax.fori_loop(0, SM_RPT, wstep, 0)
    pltpu.sync_copy(al_v, w_hbm.at[pl.ds(rbase, SM_RPT)])


def _sc_gather(ta, q0, src, dst):
    mesh = plsc.VectorSubcoreMesh(core_axis_name="c", subcore_axis_name="s",
                                  num_cores=2, num_subcores=16)
    f = pl.kernel(
        _gather_body,
        out_type=[jax.ShapeDtypeStruct((E_PAD, 2 * D), jnp.int32),
                  jax.ShapeDtypeStruct((E_PAD, D), F32)],
        mesh=mesh,
        scratch_types=[pltpu.VMEM((GCH,), jnp.int32),
                       pltpu.VMEM((GCH,), jnp.int32),
                       pltpu.VMEM((GCH, 2 * D), jnp.int32),
                       pltpu.VMEM((GCH, D), F32),
                       pltpu.SemaphoreType.DMA],
        compiler_params=pltpu.CompilerParams(needs_layout_passes=False),
    )
    return f(ta, q0, src, dst)


def _sc_softmax(alpha2d, dst2d):
    mesh = plsc.VectorSubcoreMesh(core_axis_name="c", subcore_axis_name="s",
                                  num_cores=2, num_subcores=16)
    f = pl.kernel(
        _softmax_body,
        out_type=jax.ShapeDtypeStruct((SM_ROWS, SM_W), F32),
        mesh=mesh,
        scratch_types=[pltpu.VMEM((SM_RPT, SM_W), F32),
                       pltpu.VMEM((SM_RPT, SM_W), jnp.int32),
                       pltpu.VMEM((NSEG,), F32),
                       pltpu.VMEM((640,), F32),
                       pltpu.VMEM((16,), F32),
                       pltpu.VMEM((16 * SM_NW,), F32),
                       pltpu.VMEM_SHARED((NSEG + 16 * SM_NW,), F32),
                       pltpu.SemaphoreType.DMA],
        compiler_params=pltpu.CompilerParams(needs_layout_passes=False),
    )
    return f(alpha2d, dst2d)


def _tc_pre(x, wT, b, wff0T, wq1T, bff0, bq1):
    full = lambda *dims: pl.BlockSpec(dims, lambda i: (0,) * len(dims))
    return pl.pallas_call(
        _pre_body,
        grid=(N_NODES // NB,),
        in_specs=[
            pl.BlockSpec((NB, D), lambda i: (i, 0)),
            full(5, D, D), full(5, D), full(D, D), full(D, D),
            full(1, D), full(1, D),
        ],
        out_specs=[
            pl.BlockSpec((NB, D), lambda i: (i, 0)),
            pl.BlockSpec((NB, 2 * D), lambda i: (i, 0)),
            full(D, D), full(1, D),
        ],
        out_shape=[
            jax.ShapeDtypeStruct((N_NODES, D), F32),
            jax.ShapeDtypeStruct((N_NODES, 2 * D), jnp.int32),
            jax.ShapeDtypeStruct((D, D), F32),
            jax.ShapeDtypeStruct((1, D), F32),
        ],
    )(x, wT, b, wff0T, wq1T, bff0, bq1)


def _tc_alpha(qd, g12):
    return pl.pallas_call(
        _alpha_body,
        grid=(GRID_E,),
        in_specs=[
            pl.BlockSpec((EB, D), lambda i: (i, 0)),
            pl.BlockSpec((EB, D), lambda i: (i, 0)),
        ],
        out_specs=pl.BlockSpec((OB, 128), lambda i: (i, 0)),
        out_shape=jax.ShapeDtypeStruct((SM_ROWS, SM_W), F32),
        compiler_params=pltpu.CompilerParams(
            dimension_semantics=("parallel",)),
    )(qd, g12)


def _tc_pass1(qd, g12, w0, gag, gab, wfoldT, bfold):
    full = lambda *dims: pl.BlockSpec(dims, lambda i: (0,) * len(dims))
    return pl.pallas_call(
        _pass1_body,
        grid=(GRID_E,),
        in_specs=[
            pl.BlockSpec((EB, D), lambda i: (i, 0)),
            pl.BlockSpec((EB, D), lambda i: (i, 0)),
            pl.BlockSpec((EB, D), lambda i: (i, 1)),
            pl.BlockSpec((OB, 128), lambda i: (i, 0)),
            full(1, D), full(1, D), full(D, D), full(1, D),
        ],
        out_specs=[
            pl.BlockSpec((EB, D), lambda i: (i, 0)),
            pl.BlockSpec((OB, 128), lambda i: (i, 0)),
        ],
        out_shape=[
            jax.ShapeDtypeStruct((E_PAD, D), F32),
            jax.ShapeDtypeStruct((SM_ROWS, SM_W), F32),
        ],
        compiler_params=pltpu.CompilerParams(
            dimension_semantics=("parallel",)),
    )(qd, g12, g12, w0, gag, gab, wfoldT, bfold)


def _tc_pass2(q1, g12, w1, gag, gab, wffT, bff, gfg, gfb,
              w1T, b1, w4T, b4, w5T, b5, fng, fnb, wvec, bvec):
    full = lambda *dims: pl.BlockSpec(dims, lambda i: (0,) * len(dims))
    return pl.pallas_call(
        _pass2_body,
        grid=(GRID_E,),
        in_specs=[
            pl.BlockSpec((EB, D), lambda i: (i, 0)),
            pl.BlockSpec((EB, D), lambda i: (i, 1)),
            pl.BlockSpec((OB, 128), lambda i: (i, 0)),
            full(1, D), full(1, D), full(D, D), full(1, D),
            full(1, D), full(1, D),
            full(D, 3 * D), full(1, 3 * D),
            full(3 * D, 3 * D), full(1, 3 * D),
            full(3 * D, D), full(1, D),
            full(1, D), full(1, D), full(1, D), full(1, 1),
        ],
        out_specs=pl.BlockSpec((OB, 128), lambda i: (i, 0)),
        out_shape=jax.ShapeDtypeStruct((SM_ROWS, SM_W), F32),
        compiler_params=pltpu.CompilerParams(
            dimension_semantics=("parallel",)),
    )(q1, g12, w1, gag, gab, wffT, bff, gfg, gfb,
      w1T, b1, w4T, b4, w5T, b5, fng, fnb, wvec, bvec)


def kernel(x, edge_index, Wq, bq, Wk, bk, Wv, bv, Wff, bff, ga_g, ga_b,
           gf_g, gf_b, W1, b1, W4, b4, W5, b5, Wvec, bvec, fn_g, fn_b):
    npad = E_PAD - E_EDGES
    src = jnp.concatenate([edge_index[0], jnp.zeros((npad,), jnp.int32)])
    dst = jnp.concatenate([edge_index[1], jnp.zeros((npad,), jnp.int32)])
    dst_seg = jnp.concatenate(
        [edge_index[1], jnp.full((npad,), NSEG - 1, jnp.int32)])

    wT = jnp.stack([Wq[0].T, Wk[0].T, Wv[0].T, Wk[1].T, Wv[1].T])
    bstack = jnp.stack([bq[0], bk[0], bv[0], bk[1], bv[1]])
    q0, ta, wfoldT, bfold = _tc_pre(
        x, wT, bstack, Wff[0].T, Wq[1].T, bff[0][None], bq[1][None])

    g12, qd = _sc_gather(ta, q0, src, dst)

    dst2d = dst_seg.reshape(SM_ROWS, SM_W)

    alpha0 = _tc_alpha(qd, g12)
    w0 = _sc_softmax(alpha0, dst2d)

    q1, alpha1 = _tc_pass1(qd, g12, w0,
                           ga_g[0][None], ga_b[0][None], wfoldT, bfold)
    w1 = _sc_softmax(alpha1, dst2d)

    out = _tc_pass2(q1, g12, w1,
                    ga_g[1][None], ga_b[1][None], Wff[1].T, bff[1][None],
                    gf_g[None], gf_b[None],
                    W1.T, b1[None], W4.T, b4[None], W5.T, b5[None],
                    fn_g[None], fn_b[None], Wvec, bvec[None])
    return out.reshape(E_PAD)[:E_EDGES]


# speedup vs baseline: 1.2600x; 1.2600x over previous
import functools

import jax
import jax.numpy as jnp
from jax import lax
from jax.experimental import pallas as pl
from jax.experimental.pallas import tpu as pltpu
from jax.experimental.pallas import tpu_sc as plsc

F32 = jnp.float32
D = 128
N_NODES = 10000
E_EDGES = 320000
LN_EPS = 1e-5

NB = 2000
E_PAD = 327680
EB = 2048
GRID_E = E_PAD // EB
OB = EB // 128

NWORK = 32
EPW = E_PAD // NWORK
GCH = 80
NCH = EPW // GCH

SM_W = 128
SM_ROWS = E_PAD // SM_W
SM_NW = 16
SM_RPT = SM_ROWS // SM_NW
NSEG = 10240


def _leaky(x):
    return jnp.where(x >= 0, x, 0.2 * x)


def _ln(x, g, b):
    mu = jnp.mean(x, axis=1, keepdims=True)
    xc = x - mu
    var = jnp.mean(xc * xc, axis=1, keepdims=True)
    return xc / jnp.sqrt(var + LN_EPS) * g + b


def _rnd16(x):
    b = lax.bitcast_convert_type(x, jnp.int32)
    r = b + jnp.int32(0x7FFF) + (lax.shift_right_logical(b, jnp.int32(16))
                                 & jnp.int32(1))
    return lax.shift_right_logical(r, jnp.int32(16))


def _pack2(x_lo, x_hi):
    return _rnd16(x_lo) | lax.shift_left(_rnd16(x_hi), jnp.int32(16))


def _unpack_lo(p):
    return lax.bitcast_convert_type(lax.shift_left(p, jnp.int32(16)), F32)


def _unpack_hi(p):
    return lax.bitcast_convert_type(p & jnp.int32(-65536), F32)


def _unpack_cat(p):
    return jnp.concatenate([_unpack_lo(p), _unpack_hi(p)], axis=1)


def _pre_body(x_ref, wT_ref, b_ref, wff0T_ref, wq1T_ref, bff0_ref, bq1_ref,
              q0_ref, ta_ref, wfoldT_ref, bfold_ref):
    xb = x_ref[...]

    def proj(j):
        return jnp.dot(xb, wT_ref[j], preferred_element_type=F32) + b_ref[j:j + 1]

    q0_ref[...] = proj(0)
    ta_ref[:, :D] = _pack2(proj(1), proj(2))
    ta_ref[:, D:] = _pack2(proj(3), proj(4))

    @pl.when(pl.program_id(0) == 0)
    def _():
        wfoldT_ref[...] = jnp.dot(wff0T_ref[...], wq1T_ref[...],
                                  preferred_element_type=F32)
        bfold_ref[...] = (jnp.dot(bff0_ref[...], wq1T_ref[...],
                                  preferred_element_type=F32) + bq1_ref[...])


def _col_to_rows(a):
    return jnp.sum(a.reshape(OB, 128, D), axis=2)


def _rows_to_col(w):
    return jnp.broadcast_to(w[:, :, None], (OB, 128, D)).reshape(EB, D)


def _alpha_body(qd_ref, g1_ref, alpha_ref):
    k0 = _unpack_lo(g1_ref[...])
    alpha_ref[...] = _col_to_rows(qd_ref[...] * k0)


def _pass1_body(qd_ref, g1_ref, g2_ref, w_ref, gag_ref, gab_ref,
                wfoldT_ref, bfold_ref, q1_ref, alpha_ref):
    v0 = _unpack_hi(g1_ref[...])
    attn = qd_ref[...] + _rows_to_col(w_ref[...]) * v0
    h = _ln(attn, gag_ref[...], gab_ref[...])
    q1 = (jnp.dot(h.astype(jnp.bfloat16), wfoldT_ref[...].astype(jnp.bfloat16),
                  preferred_element_type=F32) + bfold_ref[...])
    q1_ref[...] = q1
    k1 = _unpack_lo(g2_ref[...])
    alpha_ref[...] = _col_to_rows(q1 * k1)


def _pass2_body(q1_ref, vs_ref, w_ref, gag_ref, gab_ref, wffT_ref, bff_ref,
                gfg_ref, gfb_ref, w1T_ref, b1_ref, w4T_ref, b4_ref,
                w5T_ref, b5_ref, fng_ref, fnb_ref, wvec_ref, bvec_ref,
                out_ref):
    bf16 = jnp.bfloat16
    attn = q1_ref[...] + _rows_to_col(w_ref[...]) * _unpack_hi(vs_ref[...])
    h = _ln(attn, gag_ref[...], gab_ref[...])
    t = (jnp.dot(h.astype(bf16), wffT_ref[...].astype(bf16),
                 preferred_element_type=F32) + bff_ref[...])
    g = _ln(_leaky(t), gfg_ref[...], gfb_ref[...])
    a = _leaky(jnp.dot(g.astype(bf16), w1T_ref[...].astype(bf16),
                       preferred_element_type=F32) + b1_ref[...])
    a = _leaky(jnp.dot(a.astype(bf16), w4T_ref[...].astype(bf16),
                       preferred_element_type=F32) + b4_ref[...])
    a = (jnp.dot(a.astype(bf16), w5T_ref[...].astype(bf16),
                 preferred_element_type=F32) + b5_ref[...] + g)
    u = _ln(a, fng_ref[...], fnb_ref[...])
    out_ref[...] = _col_to_rows(u * wvec_ref[...]) + bvec_ref[...]


def _gather_body(ta_hbm, q0_hbm, src_hbm, dst_hbm,
                 g12_hbm, qd_hbm,
                 idx_s, idx_d, rows_a, rows_q, sem):
    wid = lax.axis_index("s") * 2 + lax.axis_index("c")
    base = wid * EPW

    def step(c, carry):
        off = base + c * GCH
        pltpu.sync_copy(src_hbm.at[pl.ds(off, GCH)], idx_s)
        pltpu.sync_copy(dst_hbm.at[pl.ds(off, GCH)], idx_d)
        cp1 = pltpu.async_copy(ta_hbm.at[idx_s], rows_a, sem)
        cp2 = pltpu.async_copy(q0_hbm.at[idx_d], rows_q, sem)
        cp1.wait()
        cp2.wait()
        pltpu.sync_copy(rows_a, g12_hbm.at[pl.ds(off, GCH)])
        pltpu.sync_copy(rows_q, qd_hbm.at[pl.ds(off, GCH)])
        return carry

    lax.fori_loop(0, NCH, step, 0)


def _softmax_body(alpha_hbm, dst_hbm, w_hbm,
                  al_v, dst_v, s_loc, zb, mxv, mx_all, s_sh, sem):
    cid = lax.axis_index("c")

    @pl.when(cid == 0)
    def _core0():
        _softmax_work(alpha_hbm, dst_hbm, w_hbm,
                      al_v, dst_v, s_loc, zb, mxv, mx_all, s_sh, sem)


def _softmax_work(alpha_hbm, dst_hbm, w_hbm,
                  al_v, dst_v, s_loc, zb, mxv, mx_all, s_sh, sem):
    sid = lax.axis_index("s")
    rbase = sid * SM_RPT
    pltpu.sync_copy(alpha_hbm.at[pl.ds(rbase, SM_RPT)], al_v)
    pltpu.sync_copy(dst_hbm.at[pl.ds(rbase, SM_RPT)], dst_v)

    def zstep(i, c):
        zb[pl.ds(i * 16, 16)] = jnp.zeros((16,), F32)
        return c
    lax.fori_loop(0, 40, zstep, 0)
    pltpu.sync_copy(zb, s_sh.at[pl.ds(sid * 640, 640)])

    def mstep(r, m):
        for k in range(SM_W // 16):
            m = jnp.maximum(m, al_v[r, pl.ds(k * 16, 16)])
        return m
    mvec = lax.fori_loop(0, SM_RPT, mstep, jnp.full((16,), -1e30, F32))
    lanes = lax.iota(jnp.int32, 16)
    dnums = lax.GatherDimensionNumbers(
        offset_dims=(), collapsed_slice_dims=(0,), start_index_map=(0,))
    for sh in (8, 4, 2, 1):
        perm = lax.gather(mvec, (lanes ^ sh)[:, None], dnums, (1,),
                          mode=lax.GatherScatterMode.PROMISE_IN_BOUNDS)
        mvec = jnp.maximum(mvec, perm)
    mxv[...] = mvec
    pltpu.sync_copy(mxv, s_sh.at[pl.ds(NSEG + sid * 16, 16)])
    plsc.subcore_barrier()

    pltpu.sync_copy(s_sh.at[pl.ds(NSEG, 16 * SM_NW)], mx_all)

    def rstep(i, m):
        return jnp.maximum(m, mx_all[pl.ds(i * 16, 16)])
    cvec = lax.fori_loop(0, SM_NW, rstep, jnp.full((16,), -1e30, F32))

    def pstep(r, c):
        for k in range(SM_W // 16):
            a = al_v[r, pl.ds(k * 16, 16)]
            al_v[r, pl.ds(k * 16, 16)] = jnp.exp(a - cvec)
        return c
    lax.fori_loop(0, SM_RPT, pstep, 0)

    def sstep(r, c):
        pltpu.sync_copy(al_v.at[r], s_sh.at[dst_v.at[r]], add=True)
        return c
    lax.fori_loop(0, SM_RPT, sstep, 0)
    plsc.subcore_barrier()

    pltpu.sync_copy(s_sh.at[pl.ds(0, NSEG)], s_loc)

    def wstep(r, c):
        for k in range(SM_W // 16):
            dd = dst_v[r, pl.ds(k * 16, 16)]
            sv = plsc.load_gather(s_loc, [dd])
            p = al_v[r, pl.ds(k * 16, 16)]
            al_v[r, pl.ds(k * 16, 16)] = p / (sv + 1e-16)
        return c
    lax.fori_loop(0, SM_RPT, wstep, 0)
    pltpu.sync_copy(al_v, w_hbm.at[pl.ds(rbase, SM_RPT)])


def _sc_gather(ta, q0, src, dst):
    mesh = plsc.VectorSubcoreMesh(core_axis_name="c", subcore_axis_name="s",
                                  num_cores=2, num_subcores=16)
    f = pl.kernel(
        _gather_body,
        out_type=[jax.ShapeDtypeStruct((E_PAD, 2 * D), jnp.int32),
                  jax.ShapeDtypeStruct((E_PAD, D), F32)],
        mesh=mesh,
        scratch_types=[pltpu.VMEM((GCH,), jnp.int32),
                       pltpu.VMEM((GCH,), jnp.int32),
                       pltpu.VMEM((GCH, 2 * D), jnp.int32),
                       pltpu.VMEM((GCH, D), F32),
                       pltpu.SemaphoreType.DMA],
        compiler_params=pltpu.CompilerParams(needs_layout_passes=False),
    )
    return f(ta, q0, src, dst)


def _sc_softmax(alpha2d, dst2d):
    mesh = plsc.VectorSubcoreMesh(core_axis_name="c", subcore_axis_name="s",
                                  num_cores=2, num_subcores=16)
    f = pl.kernel(
        _softmax_body,
        out_type=jax.ShapeDtypeStruct((SM_ROWS, SM_W), F32),
        mesh=mesh,
        scratch_types=[pltpu.VMEM((SM_RPT, SM_W), F32),
                       pltpu.VMEM((SM_RPT, SM_W), jnp.int32),
                       pltpu.VMEM((NSEG,), F32),
                       pltpu.VMEM((640,), F32),
                       pltpu.VMEM((16,), F32),
                       pltpu.VMEM((16 * SM_NW,), F32),
                       pltpu.VMEM_SHARED((NSEG + 16 * SM_NW,), F32),
                       pltpu.SemaphoreType.DMA],
        compiler_params=pltpu.CompilerParams(needs_layout_passes=False),
    )
    return f(alpha2d, dst2d)


def _tc_pre(x, wT, b, wff0T, wq1T, bff0, bq1):
    full = lambda *dims: pl.BlockSpec(dims, lambda i: (0,) * len(dims))
    return pl.pallas_call(
        _pre_body,
        grid=(N_NODES // NB,),
        in_specs=[
            pl.BlockSpec((NB, D), lambda i: (i, 0)),
            full(5, D, D), full(5, D), full(D, D), full(D, D),
            full(1, D), full(1, D),
        ],
        out_specs=[
            pl.BlockSpec((NB, D), lambda i: (i, 0)),
            pl.BlockSpec((NB, 2 * D), lambda i: (i, 0)),
            full(D, D), full(1, D),
        ],
        out_shape=[
            jax.ShapeDtypeStruct((N_NODES, D), F32),
            jax.ShapeDtypeStruct((N_NODES, 2 * D), jnp.int32),
            jax.ShapeDtypeStruct((D, D), F32),
            jax.ShapeDtypeStruct((1, D), F32),
        ],
    )(x, wT, b, wff0T, wq1T, bff0, bq1)


def _tc_alpha(qd, g12):
    return pl.pallas_call(
        _alpha_body,
        grid=(GRID_E,),
        in_specs=[
            pl.BlockSpec((EB, D), lambda i: (i, 0)),
            pl.BlockSpec((EB, D), lambda i: (i, 0)),
        ],
        out_specs=pl.BlockSpec((OB, 128), lambda i: (i, 0)),
        out_shape=jax.ShapeDtypeStruct((SM_ROWS, SM_W), F32),
        compiler_params=pltpu.CompilerParams(
            dimension_semantics=("parallel",)),
    )(qd, g12)


def _tc_pass1(qd, g12, w0, gag, gab, wfoldT, bfold):
    full = lambda *dims: pl.BlockSpec(dims, lambda i: (0,) * len(dims))
    return pl.pallas_call(
        _pass1_body,
        grid=(GRID_E,),
        in_specs=[
            pl.BlockSpec((EB, D), lambda i: (i, 0)),
            pl.BlockSpec((EB, D), lambda i: (i, 0)),
            pl.BlockSpec((EB, D), lambda i: (i, 1)),
            pl.BlockSpec((OB, 128), lambda i: (i, 0)),
            full(1, D), full(1, D), full(D, D), full(1, D),
        ],
        out_specs=[
            pl.BlockSpec((EB, D), lambda i: (i, 0)),
            pl.BlockSpec((OB, 128), lambda i: (i, 0)),
        ],
        out_shape=[
            jax.ShapeDtypeStruct((E_PAD, D), F32),
            jax.ShapeDtypeStruct((SM_ROWS, SM_W), F32),
        ],
        compiler_params=pltpu.CompilerParams(
            dimension_semantics=("parallel",)),
    )(qd, g12, g12, w0, gag, gab, wfoldT, bfold)


def _tc_pass2(q1, g12, w1, gag, gab, wffT, bff, gfg, gfb,
              w1T, b1, w4T, b4, w5T, b5, fng, fnb, wvec, bvec):
    full = lambda *dims: pl.BlockSpec(dims, lambda i: (0,) * len(dims))
    return pl.pallas_call(
        _pass2_body,
        grid=(GRID_E,),
        in_specs=[
            pl.BlockSpec((EB, D), lambda i: (i, 0)),
            pl.BlockSpec((EB, D), lambda i: (i, 1)),
            pl.BlockSpec((OB, 128), lambda i: (i, 0)),
            full(1, D), full(1, D), full(D, D), full(1, D),
            full(1, D), full(1, D),
            full(D, 3 * D), full(1, 3 * D),
            full(3 * D, 3 * D), full(1, 3 * D),
            full(3 * D, D), full(1, D),
            full(1, D), full(1, D), full(1, D), full(1, 1),
        ],
        out_specs=pl.BlockSpec((OB, 128), lambda i: (i, 0)),
        out_shape=jax.ShapeDtypeStruct((SM_ROWS, SM_W), F32),
        compiler_params=pltpu.CompilerParams(
            dimension_semantics=("parallel",)),
    )(q1, g12, w1, gag, gab, wffT, bff, gfg, gfb,
      w1T, b1, w4T, b4, w5T, b5, fng, fnb, wvec, bvec)


def kernel(x, edge_index, Wq, bq, Wk, bk, Wv, bv, Wff, bff, ga_g, ga_b,
           gf_g, gf_b, W1, b1, W4, b4, W5, b5, Wvec, bvec, fn_g, fn_b):
    npad = E_PAD - E_EDGES
    spread = (jnp.arange(npad, dtype=jnp.int32) * 7) % N_NODES
    src = jnp.concatenate([edge_index[0], spread])
    dst = jnp.concatenate([edge_index[1], spread])
    dst_seg = jnp.concatenate(
        [edge_index[1], jnp.full((npad,), NSEG - 1, jnp.int32)])

    wT = jnp.stack([Wq[0].T, Wk[0].T, Wv[0].T, Wk[1].T, Wv[1].T])
    bstack = jnp.stack([bq[0], bk[0], bv[0], bk[1], bv[1]])
    q0, ta, wfoldT, bfold = _tc_pre(
        x, wT, bstack, Wff[0].T, Wq[1].T, bff[0][None], bq[1][None])

    g12, qd = _sc_gather(ta, q0, src, dst)

    dst2d = dst_seg.reshape(SM_ROWS, SM_W)

    alpha0 = _tc_alpha(qd, g12)
    w0 = _sc_softmax(alpha0, dst2d)

    q1, alpha1 = _tc_pass1(qd, g12, w0,
                           ga_g[0][None], ga_b[0][None], wfoldT, bfold)
    w1 = _sc_softmax(alpha1, dst2d)

    out = _tc_pass2(q1, g12, w1,
                    ga_g[1][None], ga_b[1][None], Wff[1].T, bff[1][None],
                    gf_g[None], gf_b[None],
                    W1.T, b1[None], W4.T, b4[None], W5.T, b5[None],
                    fn_g[None], fn_b[None], Wvec, bvec[None])
    return out.reshape(E_PAD)[:E_EDGES]
